# Initial kernel scaffold; baseline (speedup 1.0000x reference)
import jax
import jax.numpy as jnp
from jax.experimental import pallas as pl


def kernel(dense_features, sparse_features, tables, bot_w0, bot_b0, bot_w1, bot_b1, bot_w2, bot_b2, top_w0, top_b0, top_w1, top_b1, top_w2, top_b2):
    raise NotImplementedError("write your pallas kernel here")


# speedup vs baseline: 2.1996x; 2.1996x over previous
import functools

import jax
import jax.numpy as jnp
import numpy as np
from jax import lax
from jax.experimental import pallas as pl
from jax.experimental.pallas import tpu as pltpu
from jax.experimental.pallas import tpu_sc as plsc

B = 4096
NUM_DENSE = 13
NF = 26
VOCAB = 100000
DIM = 32
NFEAT = NF + 1

NC, NS = 2, 16
NW = NC * NS
TOT = B * NF
BPW = TOT // NW
CHUNK = 128
NCHUNK = BPW // CHUNK


def _sc_gather_body(tab_hbm, idx_hbm, out_hbm, idx_v, rows_v, sem):
    wid = lax.axis_index("s") * NC + lax.axis_index("c")
    base = wid * BPW
    pltpu.sync_copy(idx_hbm.at[pl.ds(base, BPW)], idx_v)
    copies = [
        pltpu.async_copy(
            tab_hbm.at[idx_v.at[pl.ds(g * CHUNK, CHUNK)]],
            rows_v.at[pl.ds(g * CHUNK, CHUNK), :],
            sem,
        )
        for g in range(NCHUNK)
    ]
    for c in copies:
        c.wait()
    pltpu.sync_copy(rows_v, out_hbm.at[pl.ds(base, BPW)])


@functools.cache
def _sc_gather():
    return pl.kernel(
        _sc_gather_body,
        out_type=jax.ShapeDtypeStruct((TOT, DIM), jnp.float32),
        mesh=plsc.VectorSubcoreMesh(
            core_axis_name="c", subcore_axis_name="s",
            num_cores=NC, num_subcores=NS),
        scratch_types=[
            pltpu.VMEM((BPW,), jnp.int32),
            pltpu.VMEM((BPW, DIM), jnp.float32),
            pltpu.SemaphoreType.DMA,
        ],
        compiler_params=pltpu.CompilerParams(use_tc_tiling_on_sc=False),
    )

BB = 128
GRID = B // BB


def _tc_body(dense_ref, embs_ref, w0t, b0, w1t, b1, w2t, b2,
             wdt, mt, tb0, t1t, tb1, t2t, tb2, out_ref):
    xT = dense_ref[...].T
    h = jnp.maximum(jnp.dot(w0t[...], xT, preferred_element_type=jnp.float32)
                    + b0[...], 0.0)
    h = jnp.maximum(jnp.dot(w1t[...], h, preferred_element_type=jnp.float32)
                    + b1[...], 0.0)
    dT = jnp.maximum(jnp.dot(w2t[...], h, preferred_element_type=jnp.float32)
                     + b2[...], 0.0)

    eT = embs_ref[...].T
    xt = jnp.concatenate([eT, dT], axis=0)
    x3 = xt.reshape(NFEAT, DIM, BB)
    zrows = [jnp.sum(x3 * x3[i][None], axis=1) for i in range(NFEAT)]
    zt = jnp.concatenate(zrows, axis=0)

    h = jnp.dot(wdt[...], dT, preferred_element_type=jnp.float32)
    h = h + jnp.dot(mt[...], zt, preferred_element_type=jnp.float32)
    h = jnp.maximum(h + tb0[...], 0.0)
    h = jnp.maximum(jnp.dot(t1t[...], h, preferred_element_type=jnp.float32)
                    + tb1[...], 0.0)
    o = jnp.dot(t2t[...], h, preferred_element_type=jnp.float32) + tb2[...]
    out_ref[...] = jax.nn.sigmoid(o)


def _const_spec(shape):
    return pl.BlockSpec(shape, lambda i: (0,) * len(shape))


_tc_call = pl.pallas_call(
    _tc_body,
    grid=(GRID,),
    in_specs=[
        pl.BlockSpec((BB, NUM_DENSE), lambda i: (i, 0)),
        pl.BlockSpec((BB, NF * DIM), lambda i: (i, 0)),
        _const_spec((512, NUM_DENSE)), _const_spec((512, 1)),
        _const_spec((256, 512)), _const_spec((256, 1)),
        _const_spec((DIM, 256)), _const_spec((DIM, 1)),
        _const_spec((512, DIM)), _const_spec((512, NFEAT * NFEAT)),
        _const_spec((512, 1)),
        _const_spec((256, 512)), _const_spec((256, 1)),
        _const_spec((1, 256)), _const_spec((1, 1)),
    ],
    out_specs=pl.BlockSpec((1, BB), lambda i: (0, i)),
    out_shape=jax.ShapeDtypeStruct((1, B), jnp.float32),
)

_LI, _LJ = np.tril_indices(NFEAT, -1)
_M_ROWS = jnp.asarray(_LI * NFEAT + _LJ, dtype=jnp.int32)


def kernel(dense_features, sparse_features, tables,
           bot_w0, bot_b0, bot_w1, bot_b1, bot_w2, bot_b2,
           top_w0, top_b0, top_w1, top_b1, top_w2, top_b2):
    flat_tables = tables.reshape(NF * VOCAB, DIM)
    flat_idx = (sparse_features.astype(jnp.int32)
                + jnp.arange(NF, dtype=jnp.int32)[None, :] * VOCAB).reshape(-1)
    embs = _sc_gather()(flat_tables, flat_idx).reshape(B, NF * DIM)

    m = jnp.zeros((NFEAT * NFEAT, 512), jnp.float32).at[_M_ROWS].set(top_w0[DIM:])
    out = _tc_call(
        dense_features, embs,
        bot_w0.T, bot_b0[:, None], bot_w1.T, bot_b1[:, None],
        bot_w2.T, bot_b2[:, None],
        top_w0[:DIM].T, m.T, top_b0[:, None],
        top_w1.T, top_b1[:, None], top_w2.T, top_b2[:, None],
    )
    return out.reshape(-1)


# speedup vs baseline: 3.7437x; 1.7020x over previous
import functools

import jax
import jax.numpy as jnp
import numpy as np
from jax import lax
from jax.experimental import pallas as pl
from jax.experimental.pallas import tpu as pltpu
from jax.experimental.pallas import tpu_sc as plsc

B = 4096
NUM_DENSE = 13
NF = 26
VOCAB = 100000
DIM = 32
NFEAT = NF + 1

NC, NS = 2, 16
NW = NC * NS
NROWS = NF * DIM
RPW = NROWS // NW
NVEC = B // 16


def _sc_gather_body(q_hbm, idxt_hbm, out_hbm, row_v, idx_v, out_v):
    wid = lax.axis_index("s") * NC + lax.axis_index("c")

    def row_step(j, _):
        r = wid * RPW + j
        f = r // DIM
        pltpu.sync_copy(idxt_hbm.at[f], idx_v)
        pltpu.sync_copy(q_hbm.at[r], row_v)

        def gather_step(k, _):
            iv = idx_v[pl.ds(k * 16, 16)]
            out_v[pl.ds(k * 16, 16)] = plsc.load_gather(row_v, [iv])
            return 0

        lax.fori_loop(0, NVEC, gather_step, 0)
        pltpu.sync_copy(out_v, out_hbm.at[r])
        return 0

    lax.fori_loop(0, RPW, row_step, 0)


@functools.cache
def _sc_gather():
    return pl.kernel(
        _sc_gather_body,
        out_type=jax.ShapeDtypeStruct((NROWS, B), jnp.float32),
        mesh=plsc.VectorSubcoreMesh(
            core_axis_name="c", subcore_axis_name="s",
            num_cores=NC, num_subcores=NS),
        scratch_types=[
            pltpu.VMEM((VOCAB,), jnp.float32),
            pltpu.VMEM((B,), jnp.int32),
            pltpu.VMEM((B,), jnp.float32),
        ],
        compiler_params=pltpu.CompilerParams(
            use_tc_tiling_on_sc=False, needs_layout_passes=False),
    )

BB = 128
GRID = B // BB


def _tc_body(denset_ref, embst_ref, w0t, b0, w1t, b1, w2t, b2,
             wdt, mt, tb0, t1t, tb1, t2t, tb2, out_ref):
    xT = denset_ref[...]
    h = jnp.maximum(jnp.dot(w0t[...], xT, preferred_element_type=jnp.float32)
                    + b0[...], 0.0)
    h = jnp.maximum(jnp.dot(w1t[...], h, preferred_element_type=jnp.float32)
                    + b1[...], 0.0)
    dT = jnp.maximum(jnp.dot(w2t[...], h, preferred_element_type=jnp.float32)
                     + b2[...], 0.0)

    eT = embst_ref[...]
    xt = jnp.concatenate([eT, dT], axis=0)
    x3 = xt.reshape(NFEAT, DIM, BB)
    zrows = [jnp.sum(x3 * x3[i][None], axis=1) for i in range(NFEAT)]
    zt = jnp.concatenate(zrows, axis=0)

    h = jnp.dot(wdt[...], dT, preferred_element_type=jnp.float32)
    h = h + jnp.dot(mt[...], zt, preferred_element_type=jnp.float32)
    h = jnp.maximum(h + tb0[...], 0.0)
    h = jnp.maximum(jnp.dot(t1t[...], h, preferred_element_type=jnp.float32)
                    + tb1[...], 0.0)
    o = jnp.dot(t2t[...], h, preferred_element_type=jnp.float32) + tb2[...]
    out_ref[...] = jax.nn.sigmoid(o)


def _const_spec(shape):
    return pl.BlockSpec(shape, lambda i: (0,) * len(shape))


_tc_call = pl.pallas_call(
    _tc_body,
    grid=(GRID,),
    in_specs=[
        pl.BlockSpec((NUM_DENSE, BB), lambda i: (0, i)),
        pl.BlockSpec((NROWS, BB), lambda i: (0, i)),
        _const_spec((512, NUM_DENSE)), _const_spec((512, 1)),
        _const_spec((256, 512)), _const_spec((256, 1)),
        _const_spec((DIM, 256)), _const_spec((DIM, 1)),
        _const_spec((512, DIM)), _const_spec((512, NFEAT * NFEAT)),
        _const_spec((512, 1)),
        _const_spec((256, 512)), _const_spec((256, 1)),
        _const_spec((1, 256)), _const_spec((1, 1)),
    ],
    out_specs=pl.BlockSpec((1, BB), lambda i: (0, i)),
    out_shape=jax.ShapeDtypeStruct((1, B), jnp.float32),
)

_LI, _LJ = np.tril_indices(NFEAT, -1)
_M_ROWS = np.asarray(_LI * NFEAT + _LJ, dtype=np.int32)


def kernel(dense_features, sparse_features, tables,
           bot_w0, bot_b0, bot_w1, bot_b1, bot_w2, bot_b2,
           top_w0, top_b0, top_w1, top_b1, top_w2, top_b2):
    q = jnp.swapaxes(tables, 1, 2).reshape(NROWS, VOCAB)
    idxt = sparse_features.T.astype(jnp.int32)
    embst = _sc_gather()(q, idxt)

    m = jnp.zeros((NFEAT * NFEAT, 512), jnp.float32).at[_M_ROWS].set(top_w0[DIM:])
    out = _tc_call(
        dense_features.T, embst,
        bot_w0.T, bot_b0[:, None], bot_w1.T, bot_b1[:, None],
        bot_w2.T, bot_b2[:, None],
        top_w0[:DIM].T, m.T, top_b0[:, None],
        top_w1.T, top_b1[:, None], top_w2.T, top_b2[:, None],
    )
    return out.reshape(-1)


# speedup vs baseline: 10.2514x; 2.7383x over previous
import functools

import jax
import jax.numpy as jnp
import numpy as np
from jax import lax
from jax.experimental import pallas as pl
from jax.experimental.pallas import tpu as pltpu
from jax.experimental.pallas import tpu_sc as plsc

B = 4096
NUM_DENSE = 13
NF = 26
VOCAB = 100000
DIM = 32
NFEAT = NF + 1

NC, NS = 2, 16
NW = NC * NS
NROWS = NF * DIM
RPW = NROWS // NW
NVEC = B // 16


def _sc_gather_body(q_hbm, idxt_hbm, out_hbm, row_v, idx_v, out_v):
    wid = lax.axis_index("s") * NC + lax.axis_index("c")

    def row_step(j, _):
        r = wid * RPW + j
        f = r // DIM
        pltpu.sync_copy(idxt_hbm.at[f], idx_v)
        pltpu.sync_copy(q_hbm.at[r], row_v)

        def gather_step(k, _):
            iv = idx_v[pl.ds(k * 16, 16)]
            out_v[pl.ds(k * 16, 16)] = plsc.load_gather(row_v, [iv])
            return 0

        lax.fori_loop(0, NVEC, gather_step, 0)
        pltpu.sync_copy(out_v, out_hbm.at[r])
        return 0

    lax.fori_loop(0, RPW, row_step, 0)


@functools.cache
def _sc_gather():
    return pl.kernel(
        _sc_gather_body,
        out_type=jax.ShapeDtypeStruct((NROWS, B), jnp.float32),
        mesh=plsc.VectorSubcoreMesh(
            core_axis_name="c", subcore_axis_name="s",
            num_cores=NC, num_subcores=NS),
        scratch_types=[
            pltpu.VMEM((VOCAB,), jnp.float32),
            pltpu.VMEM((B,), jnp.int32),
            pltpu.VMEM((B,), jnp.float32),
        ],
        compiler_params=pltpu.CompilerParams(
            use_tc_tiling_on_sc=True, needs_layout_passes=False),
    )

BB = 128
GRID = B // BB


def _tc_body(denset_ref, embst_ref, w0t, b0, w1t, b1, w2t, b2,
             wdt, mt, tb0, t1t, tb1, t2t, tb2, out_ref):
    xT = denset_ref[...]
    h = jnp.maximum(jnp.dot(w0t[...], xT, preferred_element_type=jnp.float32)
                    + b0[...], 0.0)
    h = jnp.maximum(jnp.dot(w1t[...], h, preferred_element_type=jnp.float32)
                    + b1[...], 0.0)
    dT = jnp.maximum(jnp.dot(w2t[...], h, preferred_element_type=jnp.float32)
                     + b2[...], 0.0)

    eT = embst_ref[...]
    xt = jnp.concatenate([eT, dT], axis=0)
    x3 = xt.reshape(NFEAT, DIM, BB)
    zrows = [jnp.sum(x3 * x3[i][None], axis=1) for i in range(NFEAT)]
    zt = jnp.concatenate(zrows, axis=0)

    h = jnp.dot(wdt[...], dT, preferred_element_type=jnp.float32)
    h = h + jnp.dot(mt[...], zt, preferred_element_type=jnp.float32)
    h = jnp.maximum(h + tb0[...], 0.0)
    h = jnp.maximum(jnp.dot(t1t[...], h, preferred_element_type=jnp.float32)
                    + tb1[...], 0.0)
    o = jnp.dot(t2t[...], h, preferred_element_type=jnp.float32) + tb2[...]
    out_ref[...] = jax.nn.sigmoid(o)


def _const_spec(shape):
    return pl.BlockSpec(shape, lambda i: (0,) * len(shape))


_tc_call = pl.pallas_call(
    _tc_body,
    grid=(GRID,),
    in_specs=[
        pl.BlockSpec((NUM_DENSE, BB), lambda i: (0, i)),
        pl.BlockSpec((NROWS, BB), lambda i: (0, i)),
        _const_spec((512, NUM_DENSE)), _const_spec((512, 1)),
        _const_spec((256, 512)), _const_spec((256, 1)),
        _const_spec((DIM, 256)), _const_spec((DIM, 1)),
        _const_spec((512, DIM)), _const_spec((512, NFEAT * NFEAT)),
        _const_spec((512, 1)),
        _const_spec((256, 512)), _const_spec((256, 1)),
        _const_spec((1, 256)), _const_spec((1, 1)),
    ],
    out_specs=pl.BlockSpec((1, BB), lambda i: (0, i)),
    out_shape=jax.ShapeDtypeStruct((1, B), jnp.float32),
)

_LI, _LJ = np.tril_indices(NFEAT, -1)
_M_ROWS = np.asarray(_LI * NFEAT + _LJ, dtype=np.int32)


def kernel(dense_features, sparse_features, tables,
           bot_w0, bot_b0, bot_w1, bot_b1, bot_w2, bot_b2,
           top_w0, top_b0, top_w1, top_b1, top_w2, top_b2):
    q = jnp.swapaxes(tables, 1, 2).reshape(NROWS, VOCAB)
    idxt = sparse_features.T.astype(jnp.int32)
    embst = _sc_gather()(q, idxt)

    m = jnp.zeros((NFEAT * NFEAT, 512), jnp.float32).at[_M_ROWS].set(top_w0[DIM:])
    out = _tc_call(
        dense_features.T, embst,
        bot_w0.T, bot_b0[:, None], bot_w1.T, bot_b1[:, None],
        bot_w2.T, bot_b2[:, None],
        top_w0[:DIM].T, m.T, top_b0[:, None],
        top_w1.T, top_b1[:, None], top_w2.T, top_b2[:, None],
    )
    return out.reshape(-1)


# speedup vs baseline: 11.2131x; 1.0938x over previous
import functools

import jax
import jax.numpy as jnp
import numpy as np
from jax import lax
from jax.experimental import pallas as pl
from jax.experimental.pallas import tpu as pltpu
from jax.experimental.pallas import tpu_sc as plsc

B = 4096
NUM_DENSE = 13
NF = 26
VOCAB = 100000
DIM = 32
NFEAT = NF + 1

NC, NS = 2, 16
NW = NC * NS
NROWS = NF * DIM
RPW = NROWS // NW
NVEC = B // 16


def _sc_gather_body(q_hbm, idxt_hbm, out_hbm, row_v, idx_v, out_v):
    wid = lax.axis_index("s") * NC + lax.axis_index("c")

    def row_step(j, _):
        r = wid * RPW + j
        f = r // DIM
        pltpu.sync_copy(idxt_hbm.at[f], idx_v)
        pltpu.sync_copy(q_hbm.at[r], row_v)

        def gather_step(k, _):
            iv = idx_v[pl.ds(k * 16, 16)]
            out_v[pl.ds(k * 16, 16)] = plsc.load_gather(row_v, [iv])
            return 0

        lax.fori_loop(0, NVEC, gather_step, 0)
        pltpu.sync_copy(out_v, out_hbm.at[r])
        return 0

    lax.fori_loop(0, RPW, row_step, 0)


@functools.cache
def _sc_gather():
    return pl.kernel(
        _sc_gather_body,
        out_type=jax.ShapeDtypeStruct((NROWS, B), jnp.float32),
        mesh=plsc.VectorSubcoreMesh(
            core_axis_name="c", subcore_axis_name="s",
            num_cores=NC, num_subcores=NS),
        scratch_types=[
            pltpu.VMEM((VOCAB,), jnp.float32),
            pltpu.VMEM((B,), jnp.int32),
            pltpu.VMEM((B,), jnp.float32),
        ],
        compiler_params=pltpu.CompilerParams(
            use_tc_tiling_on_sc=True, needs_layout_passes=False),
    )

BB = 256
GRID = B // BB


def _tc_bot_body(denset_ref, w0t, b0, w1t, b1, w2t, b2, out_ref):
    xT = denset_ref[...]
    h = jnp.maximum(jnp.dot(w0t[...], xT, preferred_element_type=jnp.float32)
                    + b0[...], 0.0)
    h = jnp.maximum(jnp.dot(w1t[...], h, preferred_element_type=jnp.float32)
                    + b1[...], 0.0)
    out_ref[...] = jnp.maximum(
        jnp.dot(w2t[...], h, preferred_element_type=jnp.float32)
        + b2[...], 0.0)


_tc_bot = pl.pallas_call(
    _tc_bot_body,
    grid=(GRID,),
    in_specs=[
        pl.BlockSpec((NUM_DENSE, BB), lambda i: (0, i)),
        pl.BlockSpec((512, NUM_DENSE), lambda i: (0, 0)),
        pl.BlockSpec((512, 1), lambda i: (0, 0)),
        pl.BlockSpec((256, 512), lambda i: (0, 0)),
        pl.BlockSpec((256, 1), lambda i: (0, 0)),
        pl.BlockSpec((DIM, 256), lambda i: (0, 0)),
        pl.BlockSpec((DIM, 1), lambda i: (0, 0)),
    ],
    out_specs=pl.BlockSpec((DIM, BB), lambda i: (0, i)),
    out_shape=jax.ShapeDtypeStruct((DIM, B), jnp.float32),
)


def _tc_body(dt_ref, embst_ref, wdt, mt, tb0, t1t, tb1, t2t, tb2, out_ref):
    dT = dt_ref[...]
    eT = embst_ref[...]
    xt = jnp.concatenate([eT, dT], axis=0)
    x3 = xt.reshape(NFEAT, DIM, BB)
    zrows = [jnp.sum(x3 * x3[i][None], axis=1) for i in range(NFEAT)]
    zt = jnp.concatenate(zrows, axis=0)

    h = jnp.dot(wdt[...], dT, preferred_element_type=jnp.float32)
    h = h + jnp.dot(mt[...], zt, preferred_element_type=jnp.float32)
    h = jnp.maximum(h + tb0[...], 0.0)
    h = jnp.maximum(jnp.dot(t1t[...], h, preferred_element_type=jnp.float32)
                    + tb1[...], 0.0)
    o = jnp.dot(t2t[...], h, preferred_element_type=jnp.float32) + tb2[...]
    out_ref[...] = jax.nn.sigmoid(o)


def _const_spec(shape):
    return pl.BlockSpec(shape, lambda i: (0,) * len(shape))


_tc_call = pl.pallas_call(
    _tc_body,
    grid=(GRID,),
    in_specs=[
        pl.BlockSpec((DIM, BB), lambda i: (0, i)),
        pl.BlockSpec((NROWS, BB), lambda i: (0, i)),
        _const_spec((512, DIM)), _const_spec((512, NFEAT * NFEAT)),
        _const_spec((512, 1)),
        _const_spec((256, 512)), _const_spec((256, 1)),
        _const_spec((1, 256)), _const_spec((1, 1)),
    ],
    out_specs=pl.BlockSpec((1, BB), lambda i: (0, i)),
    out_shape=jax.ShapeDtypeStruct((1, B), jnp.float32),
)

_LI, _LJ = np.tril_indices(NFEAT, -1)
_M_ROWS = np.asarray(_LI * NFEAT + _LJ, dtype=np.int32)


def kernel(dense_features, sparse_features, tables,
           bot_w0, bot_b0, bot_w1, bot_b1, bot_w2, bot_b2,
           top_w0, top_b0, top_w1, top_b1, top_w2, top_b2):
    q = jnp.swapaxes(tables, 1, 2).reshape(NROWS, VOCAB)
    idxt = sparse_features.T.astype(jnp.int32)
    embst = _sc_gather()(q, idxt)

    dt = _tc_bot(
        dense_features.T,
        bot_w0.T, bot_b0[:, None], bot_w1.T, bot_b1[:, None],
        bot_w2.T, bot_b2[:, None],
    )
    m = jnp.zeros((NFEAT * NFEAT, 512), jnp.float32).at[_M_ROWS].set(top_w0[DIM:])
    out = _tc_call(
        dt, embst,
        top_w0[:DIM].T, m.T, top_b0[:, None],
        top_w1.T, top_b1[:, None], top_w2.T, top_b2[:, None],
    )
    return out.reshape(-1)


# speedup vs baseline: 12.9645x; 1.1562x over previous
import functools

import jax
import jax.numpy as jnp
import numpy as np
from jax import lax
from jax.experimental import pallas as pl
from jax.experimental.pallas import tpu as pltpu
from jax.experimental.pallas import tpu_sc as plsc

B = 4096
NUM_DENSE = 13
NF = 26
VOCAB = 100000
DIM = 32
NFEAT = NF + 1

NC, NS = 2, 16
NW = NC * NS
NROWS = NF * DIM
RPW = NROWS // NW
NVEC = B // 16
SEG0 = 50048
SEG1 = VOCAB - SEG0


def _sc_gather_body(q_hbm, idxt_hbm, out_hbm,
                    buf0, buf1, idx0, idx1, o0, o1, ssem0, ssem1, isem, osem):
    wid = lax.axis_index("s") * NC + lax.axis_index("c")
    base = wid * RPW

    def start_h0(r):
        rr = jnp.minimum(r, NROWS - 1)
        pltpu.async_copy(q_hbm.at[rr, pl.ds(0, SEG0)], buf0, ssem0)

    def start_h1(r):
        rr = jnp.minimum(r, NROWS - 1)
        pltpu.async_copy(q_hbm.at[rr, pl.ds(SEG0, SEG1)], buf1, ssem1)

    def start_idx(r, idx_v):
        rr = jnp.minimum(r, NROWS - 1)
        pltpu.async_copy(idxt_hbm.at[rr // DIM], idx_v, isem)

    def wait_h0():
        pltpu.make_async_copy(q_hbm.at[0, pl.ds(0, SEG0)], buf0, ssem0).wait()

    def wait_h1():
        pltpu.make_async_copy(q_hbm.at[0, pl.ds(SEG0, SEG1)], buf1, ssem1).wait()

    def wait_idx(idx_v):
        pltpu.make_async_copy(idxt_hbm.at[0], idx_v, isem).wait()

    def wait_out(out_v):
        pltpu.make_async_copy(out_v, out_hbm.at[0], osem).wait()

    start_idx(base, idx0)
    start_h0(base)
    start_h1(base)

    def do_row(r, j, idx_v, idx_n, out_v, first):
        wait_idx(idx_v)
        start_idx(r + 1, idx_n)
        @pl.when(jnp.logical_not(first))
        def _():
            wait_out(out_v)

        wait_h0()

        def g0(k, _):
            iv = idx_v[pl.ds(k * 16, 16)]
            loc = jnp.minimum(iv, SEG0 - 1)
            out_v[pl.ds(k * 16, 16)] = plsc.load_gather(buf0, [loc])
            return 0

        lax.fori_loop(0, NVEC, g0, 0)
        start_h0(r + 1)

        wait_h1()

        def g1(k, _):
            iv = idx_v[pl.ds(k * 16, 16)]
            loc = jnp.maximum(iv - SEG0, 0)
            g = plsc.load_gather(buf1, [loc])
            prev = out_v[pl.ds(k * 16, 16)]
            out_v[pl.ds(k * 16, 16)] = jnp.where(iv >= SEG0, g, prev)
            return 0

        lax.fori_loop(0, NVEC, g1, 0)
        start_h1(r + 1)
        pltpu.async_copy(out_v, out_hbm.at[r], osem)

    def pair_step(j, _):
        ra = base + 2 * j
        do_row(ra, j, idx0, idx1, o0, j == 0)
        do_row(ra + 1, j, idx1, idx0, o1, j == 0)
        return 0

    lax.fori_loop(0, RPW // 2, pair_step, 0)
    wait_h0()
    wait_h1()
    wait_idx(idx0)
    wait_out(o0)
    wait_out(o1)


@functools.cache
def _sc_gather():
    return pl.kernel(
        _sc_gather_body,
        out_type=jax.ShapeDtypeStruct((NROWS, B), jnp.float32),
        mesh=plsc.VectorSubcoreMesh(
            core_axis_name="c", subcore_axis_name="s",
            num_cores=NC, num_subcores=NS),
        scratch_types=[
            pltpu.VMEM((SEG0,), jnp.float32),
            pltpu.VMEM((SEG1,), jnp.float32),
            pltpu.VMEM((B,), jnp.int32),
            pltpu.VMEM((B,), jnp.int32),
            pltpu.VMEM((B,), jnp.float32),
            pltpu.VMEM((B,), jnp.float32),
            pltpu.SemaphoreType.DMA,
            pltpu.SemaphoreType.DMA,
            pltpu.SemaphoreType.DMA,
            pltpu.SemaphoreType.DMA,
        ],
        compiler_params=pltpu.CompilerParams(
            use_tc_tiling_on_sc=True, needs_layout_passes=False),
    )

BB = 256
GRID = B // BB


def _tc_bot_body(denset_ref, w0t, b0, w1t, b1, w2t, b2, out_ref):
    xT = denset_ref[...]
    h = jnp.maximum(jnp.dot(w0t[...], xT, preferred_element_type=jnp.float32)
                    + b0[...], 0.0)
    h = jnp.maximum(jnp.dot(w1t[...], h, preferred_element_type=jnp.float32)
                    + b1[...], 0.0)
    out_ref[...] = jnp.maximum(
        jnp.dot(w2t[...], h, preferred_element_type=jnp.float32)
        + b2[...], 0.0)


_tc_bot = pl.pallas_call(
    _tc_bot_body,
    grid=(GRID,),
    in_specs=[
        pl.BlockSpec((NUM_DENSE, BB), lambda i: (0, i)),
        pl.BlockSpec((512, NUM_DENSE), lambda i: (0, 0)),
        pl.BlockSpec((512, 1), lambda i: (0, 0)),
        pl.BlockSpec((256, 512), lambda i: (0, 0)),
        pl.BlockSpec((256, 1), lambda i: (0, 0)),
        pl.BlockSpec((DIM, 256), lambda i: (0, 0)),
        pl.BlockSpec((DIM, 1), lambda i: (0, 0)),
    ],
    out_specs=pl.BlockSpec((DIM, BB), lambda i: (0, i)),
    out_shape=jax.ShapeDtypeStruct((DIM, B), jnp.float32),
)


def _tc_body(dt_ref, embst_ref, wdt, mt, tb0, t1t, tb1, t2t, tb2, out_ref):
    dT = dt_ref[...]
    eT = embst_ref[...]
    xt = jnp.concatenate([eT, dT], axis=0)
    x3 = xt.reshape(NFEAT, DIM, BB)
    zrows = [jnp.sum(x3 * x3[i][None], axis=1) for i in range(NFEAT)]
    zt = jnp.concatenate(zrows, axis=0)

    h = jnp.dot(wdt[...], dT, preferred_element_type=jnp.float32)
    h = h + jnp.dot(mt[...], zt, preferred_element_type=jnp.float32)
    h = jnp.maximum(h + tb0[...], 0.0)
    h = jnp.maximum(jnp.dot(t1t[...], h, preferred_element_type=jnp.float32)
                    + tb1[...], 0.0)
    o = jnp.dot(t2t[...], h, preferred_element_type=jnp.float32) + tb2[...]
    out_ref[...] = jax.nn.sigmoid(o)


def _const_spec(shape):
    return pl.BlockSpec(shape, lambda i: (0,) * len(shape))


_tc_call = pl.pallas_call(
    _tc_body,
    grid=(GRID,),
    in_specs=[
        pl.BlockSpec((DIM, BB), lambda i: (0, i)),
        pl.BlockSpec((NROWS, BB), lambda i: (0, i)),
        _const_spec((512, DIM)), _const_spec((512, NFEAT * NFEAT)),
        _const_spec((512, 1)),
        _const_spec((256, 512)), _const_spec((256, 1)),
        _const_spec((1, 256)), _const_spec((1, 1)),
    ],
    out_specs=pl.BlockSpec((1, BB), lambda i: (0, i)),
    out_shape=jax.ShapeDtypeStruct((1, B), jnp.float32),
)

_LI, _LJ = np.tril_indices(NFEAT, -1)
_M_ROWS = np.asarray(_LI * NFEAT + _LJ, dtype=np.int32)


def kernel(dense_features, sparse_features, tables,
           bot_w0, bot_b0, bot_w1, bot_b1, bot_w2, bot_b2,
           top_w0, top_b0, top_w1, top_b1, top_w2, top_b2):
    q = jnp.swapaxes(tables, 1, 2).reshape(NROWS, VOCAB)
    idxt = sparse_features.T.astype(jnp.int32)
    embst = _sc_gather()(q, idxt)

    dt = _tc_bot(
        dense_features.T,
        bot_w0.T, bot_b0[:, None], bot_w1.T, bot_b1[:, None],
        bot_w2.T, bot_b2[:, None],
    )
    m = jnp.zeros((NFEAT * NFEAT, 512), jnp.float32).at[_M_ROWS].set(top_w0[DIM:])
    out = _tc_call(
        dt, embst,
        top_w0[:DIM].T, m.T, top_b0[:, None],
        top_w1.T, top_b1[:, None], top_w2.T, top_b2[:, None],
    )
    return out.reshape(-1)
